# baseline (speedup 1.0000x reference)
import functools

import jax
import jax.numpy as jnp
from jax import lax
from jax.experimental import pallas as pl
from jax.experimental.pallas import tpu as pltpu
from jax.experimental.pallas import tpu_sc as plsc

ROWS_BLK = 1024
K_BLK = 2048


def _argmin_body(k_blk, z2_ref, cb_ref, idx_ref, minv_ref, mini_ref):
    j = pl.program_id(1)
    nj = pl.num_programs(1)
    z2 = z2_ref[...]
    c = cb_ref[...]
    a = 0.25 * jnp.sum(z2 * z2, axis=1, keepdims=True)
    b = jnp.sum(c * c, axis=1)
    m = lax.dot_general(z2, c, (((1,), (1,)), ((), ())),
                        preferred_element_type=jnp.float32)
    d = (a + b[None, :]) + m
    lmin = jnp.min(d, axis=1)
    iota = lax.broadcasted_iota(jnp.int32, d.shape, 1)
    lidx = jnp.min(jnp.where(d == lmin[:, None], iota, jnp.int32(2**30)),
                   axis=1) + j * k_blk

    @pl.when(j == 0)
    def _():
        minv_ref[...] = lmin
        mini_ref[...] = lidx

    @pl.when(j > 0)
    def _():
        better = lmin < minv_ref[...]
        mini_ref[...] = jnp.where(better, lidx, mini_ref[...])
        minv_ref[...] = jnp.where(better, lmin, minv_ref[...])

    @pl.when(j == nj - 1)
    def _():
        idx_ref[...] = mini_ref[...]


def _argmin_call(zf, codebook, interpret=False):
    n, c = zf.shape
    k = codebook.shape[0]
    body = functools.partial(_argmin_body, K_BLK)
    return pl.pallas_call(
        body,
        grid=(n // ROWS_BLK, k // K_BLK),
        in_specs=[
            pl.BlockSpec((ROWS_BLK, c), lambda i, j: (i, 0)),
            pl.BlockSpec((K_BLK, c), lambda i, j: (j, 0)),
        ],
        out_specs=pl.BlockSpec((ROWS_BLK,), lambda i, j: (i,)),
        out_shape=jax.ShapeDtypeStruct((n,), jnp.int32),
        scratch_shapes=[
            pltpu.VMEM((ROWS_BLK,), jnp.float32),
            pltpu.VMEM((ROWS_BLK,), jnp.int32),
        ],
        interpret=interpret,
    )(zf, codebook)


def _gather_call(codebook, idx):
    n = idx.shape[0]
    k, d = codebook.shape
    info = plsc.get_sparse_core_info()
    nw = info.num_cores * info.num_subcores
    b_per_w = n // nw
    mesh = plsc.VectorSubcoreMesh(core_axis_name="c", subcore_axis_name="s")

    @functools.partial(
        pl.kernel,
        mesh=mesh,
        out_type=jax.ShapeDtypeStruct((n, d), jnp.float32),
        scratch_types=[
            pltpu.VMEM((b_per_w,), jnp.int32),
            pltpu.VMEM((b_per_w, d), jnp.float32),
            pltpu.SemaphoreType.DMA,
        ],
    )
    def gather(table_hbm, idx_hbm, out_hbm, idx_v, rows_v, sem):
        wid = lax.axis_index("s") * info.num_cores + lax.axis_index("c")
        base = wid * b_per_w
        pltpu.sync_copy(idx_hbm.at[pl.ds(base, b_per_w)], idx_v)
        pltpu.async_copy(table_hbm.at[idx_v], rows_v, sem).wait()
        pltpu.sync_copy(rows_v, out_hbm.at[pl.ds(base, b_per_w)])

    return gather(codebook, idx)


def kernel(z_e, codebook):
    b, c, h, w = z_e.shape
    n = b * h * w
    zf = jnp.transpose(z_e.reshape(b, c, h * w), (0, 2, 1)).reshape(n, c)
    idx = _argmin_call(zf * -2.0, codebook)
    quant = _gather_call(codebook, idx)
    quantized = jnp.transpose(
        quant.reshape(b, h * w, c), (0, 2, 1)).reshape(b, c, h, w)
    return quantized, idx.reshape(b, h * w)


# speedup vs baseline: 1.2448x; 1.2448x over previous
import functools

import jax
import jax.numpy as jnp
from jax import lax
from jax.experimental import pallas as pl
from jax.experimental.pallas import tpu as pltpu
from jax.experimental.pallas import tpu_sc as plsc

ROWS_BLK = 1024
K_BLK = 2048


def _argmin_body(k_blk, z2_ref, cb_ref, idx_ref, minv_ref, mini_ref):
    j = pl.program_id(1)
    nj = pl.num_programs(1)
    z2 = z2_ref[...]
    c = cb_ref[...]
    a = 0.25 * jnp.sum(z2 * z2, axis=1, keepdims=True)
    b = jnp.sum(c * c, axis=1)
    m = lax.dot_general(z2, c, (((1,), (1,)), ((), ())),
                        preferred_element_type=jnp.float32)
    d = (a + b[None, :]) + m
    lmin = jnp.min(d, axis=1, keepdims=True)
    iota = lax.broadcasted_iota(jnp.int32, d.shape, 1)
    lidx = jnp.min(jnp.where(d == lmin, iota, jnp.int32(2**30)),
                   axis=1, keepdims=True) + j * k_blk

    @pl.when(j == 0)
    def _():
        minv_ref[...] = lmin
        mini_ref[...] = lidx

    @pl.when(j > 0)
    def _():
        better = lmin < minv_ref[...]
        mini_ref[...] = jnp.where(better, lidx, mini_ref[...])
        minv_ref[...] = jnp.where(better, lmin, minv_ref[...])

    @pl.when(j == nj - 1)
    def _():
        idx_ref[...] = mini_ref[...]


def _argmin_call(zf, codebook, interpret=False):
    n, c = zf.shape
    k = codebook.shape[0]
    body = functools.partial(_argmin_body, K_BLK)
    return pl.pallas_call(
        body,
        grid=(n // ROWS_BLK, k // K_BLK),
        in_specs=[
            pl.BlockSpec((ROWS_BLK, c), lambda i, j: (i, 0)),
            pl.BlockSpec((K_BLK, c), lambda i, j: (j, 0)),
        ],
        out_specs=pl.BlockSpec((ROWS_BLK, 1), lambda i, j: (i, 0)),
        out_shape=jax.ShapeDtypeStruct((n, 1), jnp.int32),
        scratch_shapes=[
            pltpu.VMEM((ROWS_BLK, 1), jnp.float32),
            pltpu.VMEM((ROWS_BLK, 1), jnp.int32),
        ],
        interpret=interpret,
    )(zf, codebook).reshape(n)


def _gather_call(codebook, idx):
    n = idx.shape[0]
    k, d = codebook.shape
    info = plsc.get_sparse_core_info()
    nw = info.num_cores * info.num_subcores
    b_per_w = n // nw
    mesh = plsc.VectorSubcoreMesh(core_axis_name="c", subcore_axis_name="s")

    @functools.partial(
        pl.kernel,
        mesh=mesh,
        out_type=jax.ShapeDtypeStruct((n, d), jnp.float32),
        scratch_types=[
            pltpu.VMEM((b_per_w,), jnp.int32),
            pltpu.VMEM((b_per_w, d), jnp.float32),
            pltpu.SemaphoreType.DMA,
        ],
    )
    def gather(table_hbm, idx_hbm, out_hbm, idx_v, rows_v, sem):
        wid = lax.axis_index("s") * info.num_cores + lax.axis_index("c")
        base = wid * b_per_w
        pltpu.sync_copy(idx_hbm.at[pl.ds(base, b_per_w)], idx_v)
        pltpu.async_copy(table_hbm.at[idx_v], rows_v, sem).wait()
        pltpu.sync_copy(rows_v, out_hbm.at[pl.ds(base, b_per_w)])

    return gather(codebook, idx)


def kernel(z_e, codebook):
    b, c, h, w = z_e.shape
    n = b * h * w
    zf = jnp.transpose(z_e.reshape(b, c, h * w), (0, 2, 1)).reshape(n, c)
    idx = _argmin_call(zf * -2.0, codebook)
    quant = _gather_call(codebook, idx)
    quantized = jnp.transpose(
        quant.reshape(b, h * w, c), (0, 2, 1)).reshape(b, c, h, w)
    return quantized, idx.reshape(b, h * w)
